# baseline (speedup 1.0000x reference)
import jax
import jax.numpy as jnp
import numpy as np
from jax.experimental import pallas as pl
from jax.experimental.pallas import tpu as pltpu

_N, _F, _H, _K = 10000, 128, 512, 16

_RY = 2000
_RI = 400
_I = _N // _RI
_R2 = 400
_I2 = _N // _R2

_PREC = jax.lax.Precision.HIGHEST


def _y_kernel(feat_ref, wg_ref, y_ref):
    y_ref[...] = jnp.dot(feat_ref[...], wg_ref[...],
                         preferred_element_type=jnp.float32, precision=_PREC)


def _assign_kernel(gn_ref, y_ref, bg_ref, wc_ref, bc_ref,
                   s_ref, cs_ref, cs_scr):
    i = pl.program_id(0)
    z = jnp.dot(gn_ref[...], y_ref[...],
                preferred_element_type=jnp.float32, precision=_PREC)
    zb = z + bg_ref[...]
    alpha = 1.6732632423543772
    scale = 1.0507009873554805
    gnn = scale * jnp.where(zb > 0, zb, alpha * (jnp.exp(zb) - 1.0))
    logits = jnp.dot(gnn, wc_ref[...],
                     preferred_element_type=jnp.float32,
                     precision=_PREC) + bc_ref[...]
    m = jnp.max(logits, axis=1, keepdims=True)
    e = jnp.exp(logits - m)
    s = e / jnp.sum(e, axis=1, keepdims=True)
    s_ref[...] = s

    @pl.when(i == 0)
    def _():
        cs_scr[...] = jnp.zeros_like(cs_scr)

    cs_scr[...] += jnp.sum(s, axis=0, keepdims=True)

    @pl.when(i == _I - 1)
    def _():
        cs_ref[...] = cs_scr[...]


def _loss_kernel(graph_ref, s_ref, cs_ref, loss_ref,
                 d_scr, tr_scr):
    i = pl.program_id(0)

    @pl.when(i == 0)
    def _():
        tr_scr[...] = jnp.zeros_like(tr_scr)
        d_scr[...] = jnp.zeros_like(d_scr)

    a = graph_ref[...]
    s_i = s_ref[pl.ds(i * _R2, _R2), :]

    p = jnp.dot(a.astype(jnp.bfloat16), s_ref[...].astype(jnp.bfloat16),
                preferred_element_type=jnp.float32)
    tr_scr[...] += jnp.sum(p * s_i).reshape(1, 1)

    d_scr[...] += jnp.sum((a != 0.0).astype(jnp.float32), axis=0,
                          keepdims=True)

    @pl.when(i == _I2 - 1)
    def _():
        d = d_scr[...]
        v = jnp.dot(d, s_ref[...],
                    preferred_element_type=jnp.float32, precision=_PREC)
        ne = jnp.sum(d)
        tr = jnp.sum(tr_scr[...])
        tr_norm = jnp.sum(v * v) / 2.0 / ne
        spectral = -(tr - tr_norm) / 2.0 / ne
        cs = cs_ref[...]
        cluster = (jnp.sqrt(jnp.sum(cs * cs)) / _N
                   * np.sqrt(float(_K)) - 1.0)
        loss_ref[...] = (spectral + cluster).reshape(1, 1)


def kernel(features, graph, graph_normalised, edge_attr,
           W_gcn, b_gcn, W_cls, b_cls):
    del edge_attr
    bg = b_gcn.reshape(1, _H)
    bc = b_cls.reshape(1, _K)

    y = pl.pallas_call(
        _y_kernel,
        grid=(_N // _RY,),
        in_specs=[
            pl.BlockSpec((_RY, _F), lambda i: (i, 0)),
            pl.BlockSpec((_F, _H), lambda i: (0, 0)),
        ],
        out_specs=pl.BlockSpec((_RY, _H), lambda i: (i, 0)),
        out_shape=jax.ShapeDtypeStruct((_N, _H), jnp.float32),
    )(features, W_gcn)

    s, cs = pl.pallas_call(
        _assign_kernel,
        grid=(_I,),
        in_specs=[
            pl.BlockSpec((_RI, _N), lambda i: (i, 0)),
            pl.BlockSpec((_N, _H), lambda i: (0, 0)),
            pl.BlockSpec((1, _H), lambda i: (0, 0)),
            pl.BlockSpec((_H, _K), lambda i: (0, 0)),
            pl.BlockSpec((1, _K), lambda i: (0, 0)),
        ],
        out_specs=[
            pl.BlockSpec((_RI, _K), lambda i: (i, 0)),
            pl.BlockSpec((1, _K), lambda i: (0, 0)),
        ],
        out_shape=[
            jax.ShapeDtypeStruct((_N, _K), jnp.float32),
            jax.ShapeDtypeStruct((1, _K), jnp.float32),
        ],
        scratch_shapes=[
            pltpu.VMEM((1, _K), jnp.float32),
        ],
    )(graph_normalised, y, bg, W_cls, bc)

    loss = pl.pallas_call(
        _loss_kernel,
        grid=(_I2,),
        in_specs=[
            pl.BlockSpec((_R2, _N), lambda i: (i, 0)),
            pl.BlockSpec((_N, _K), lambda i: (0, 0)),
            pl.BlockSpec((1, _K), lambda i: (0, 0)),
        ],
        out_specs=pl.BlockSpec((1, 1), lambda i: (0, 0)),
        out_shape=jax.ShapeDtypeStruct((1, 1), jnp.float32),
        scratch_shapes=[
            pltpu.VMEM((1, _N), jnp.float32),
            pltpu.VMEM((1, 1), jnp.float32),
        ],
    )(graph, s, cs)

    return loss[0, 0]


# speedup vs baseline: 3.0926x; 3.0926x over previous
import jax
import jax.numpy as jnp
import numpy as np
from jax.experimental import pallas as pl
from jax.experimental.pallas import tpu as pltpu

_N, _F, _H, _K = 10000, 128, 512, 16

_RY = 2000
_RI = 400
_I = _N // _RI
_R2 = 400
_I2 = _N // _R2


def _bdot(a, b):
    return jnp.dot(a.astype(jnp.bfloat16), b.astype(jnp.bfloat16),
                   preferred_element_type=jnp.float32)


def _y_kernel(feat_ref, wg_ref, y_ref):
    y_ref[...] = _bdot(feat_ref[...], wg_ref[...]).astype(jnp.bfloat16)


def _assign_kernel(gn_ref, y_ref, bg_ref, wc_ref, bc_ref, s_ref):
    z = jnp.dot(gn_ref[...].astype(jnp.bfloat16), y_ref[...],
                preferred_element_type=jnp.float32)
    zb = z + bg_ref[...]
    alpha = 1.6732632423543772
    scale = 1.0507009873554805
    gnn = scale * jnp.where(zb > 0, zb, alpha * (jnp.exp(zb) - 1.0))
    logits = _bdot(gnn, wc_ref[...]) + bc_ref[...]
    m = jnp.max(logits, axis=1, keepdims=True)
    e = jnp.exp(logits - m)
    s_ref[...] = e / jnp.sum(e, axis=1, keepdims=True)


def _loss_kernel(graph_ref, s_ref, loss_ref, d_scr, tr_scr):
    i = pl.program_id(0)

    @pl.when(i == 0)
    def _():
        tr_scr[...] = jnp.zeros_like(tr_scr)
        d_scr[...] = jnp.zeros_like(d_scr)

    a = graph_ref[...]
    s_i = s_ref[pl.ds(i * _R2, _R2), :]

    p = _bdot(a, s_ref[...])
    tr_scr[...] += jnp.sum(p * s_i).reshape(1, 1)

    d_scr[...] += jnp.sum((a != 0.0).astype(jnp.float32), axis=0,
                          keepdims=True)

    @pl.when(i == _I2 - 1)
    def _():
        d = d_scr[...]
        v = _bdot(d, s_ref[...])
        ne = jnp.sum(d)
        tr = jnp.sum(tr_scr[...])
        tr_norm = jnp.sum(v * v) / 2.0 / ne
        spectral = -(tr - tr_norm) / 2.0 / ne
        cs = jnp.sum(s_ref[...], axis=0, keepdims=True)
        cluster = (jnp.sqrt(jnp.sum(cs * cs)) / _N
                   * np.sqrt(float(_K)) - 1.0)
        loss_ref[...] = (spectral + cluster).reshape(1, 1)


def kernel(features, graph, graph_normalised, edge_attr,
           W_gcn, b_gcn, W_cls, b_cls):
    del edge_attr
    bg = b_gcn.reshape(1, _H)
    bc = b_cls.reshape(1, _K)

    y = pl.pallas_call(
        _y_kernel,
        grid=(_N // _RY,),
        in_specs=[
            pl.BlockSpec((_RY, _F), lambda i: (i, 0)),
            pl.BlockSpec((_F, _H), lambda i: (0, 0)),
        ],
        out_specs=pl.BlockSpec((_RY, _H), lambda i: (i, 0)),
        out_shape=jax.ShapeDtypeStruct((_N, _H), jnp.bfloat16),
    )(features, W_gcn)

    s = pl.pallas_call(
        _assign_kernel,
        grid=(_I,),
        in_specs=[
            pl.BlockSpec((_RI, _N), lambda i: (i, 0)),
            pl.BlockSpec((_N, _H), lambda i: (0, 0)),
            pl.BlockSpec((1, _H), lambda i: (0, 0)),
            pl.BlockSpec((_H, _K), lambda i: (0, 0)),
            pl.BlockSpec((1, _K), lambda i: (0, 0)),
        ],
        out_specs=pl.BlockSpec((_RI, _K), lambda i: (i, 0)),
        out_shape=jax.ShapeDtypeStruct((_N, _K), jnp.float32),
    )(graph_normalised, y, bg, W_cls, bc)

    loss = pl.pallas_call(
        _loss_kernel,
        grid=(_I2,),
        in_specs=[
            pl.BlockSpec((_R2, _N), lambda i: (i, 0)),
            pl.BlockSpec((_N, _K), lambda i: (0, 0)),
        ],
        out_specs=pl.BlockSpec((1, 1), lambda i: (0, 0)),
        out_shape=jax.ShapeDtypeStruct((1, 1), jnp.float32),
        scratch_shapes=[
            pltpu.VMEM((1, _N), jnp.float32),
            pltpu.VMEM((1, 1), jnp.float32),
        ],
    )(graph, s)

    return loss[0, 0]
